# baseline (speedup 1.0000x reference)
import functools

import jax
import jax.numpy as jnp
from jax import lax
from jax.experimental import pallas as pl
from jax.experimental.pallas import tpu as pltpu
from jax.experimental.pallas import tpu_sc as plsc

_NC = 2
_NS = 16
_NW = _NC * _NS
_CHUNK = 64


def _sc_gather_pair(exe_idx, skill_idx, exe_table, skill_table):
    n = exe_idx.shape[0]
    d = exe_table.shape[1]
    per_w = n // _NW
    n_chunks = per_w // _CHUNK
    mesh = plsc.VectorSubcoreMesh(
        core_axis_name="c", subcore_axis_name="s",
        num_cores=_NC, num_subcores=_NS,
    )

    @functools.partial(
        pl.kernel,
        mesh=mesh,
        out_type=[
            jax.ShapeDtypeStruct((n, d), jnp.float32),
            jax.ShapeDtypeStruct((n, d), jnp.float32),
        ],
        scratch_types=[
            pltpu.VMEM((_CHUNK,), jnp.int32),
            pltpu.VMEM((_CHUNK,), jnp.int32),
            pltpu.VMEM((_CHUNK, d), jnp.float32),
            pltpu.VMEM((_CHUNK, d), jnp.float32),
            pltpu.SemaphoreType.DMA,
            pltpu.SemaphoreType.DMA,
        ],
    )
    def gather_kernel(exe_idx_hbm, skill_idx_hbm, exe_tab_hbm, skill_tab_hbm,
                      out_exe_hbm, out_skill_hbm,
                      idx_e, idx_s, rows_e, rows_s, sem_e, sem_s):
        wid = lax.axis_index("s") * _NC + lax.axis_index("c")
        base = wid * per_w

        def body(c, carry):
            off = base + c * _CHUNK
            pltpu.sync_copy(exe_idx_hbm.at[pl.ds(off, _CHUNK)], idx_e)
            pltpu.sync_copy(skill_idx_hbm.at[pl.ds(off, _CHUNK)], idx_s)
            cp_e = pltpu.async_copy(exe_tab_hbm.at[idx_e], rows_e, sem_e)
            cp_s = pltpu.async_copy(skill_tab_hbm.at[idx_s], rows_s, sem_s)
            cp_e.wait()
            cp_s.wait()
            pltpu.sync_copy(rows_e, out_exe_hbm.at[pl.ds(off, _CHUNK)])
            pltpu.sync_copy(rows_s, out_skill_hbm.at[pl.ds(off, _CHUNK)])
            return carry

        lax.fori_loop(0, n_chunks, body, 0)

    return gather_kernel(exe_idx, skill_idx, exe_table, skill_table)


def _tc_body(in_nlp, out_nlp, exe_id, skill_id, r_id, et, gexe, gskill,
             pos, W, b, etW, etb, resp, enc_o, dec_o, out_o):
    Wv = W[...]
    bv = b[...]
    posv = pos[...]
    ids = (exe_id[...] + skill_id[...]).astype(jnp.float32)
    enc_o[...] = (
        jnp.dot(in_nlp[...], Wv, preferred_element_type=jnp.float32)
        + bv + ids + posv
    )
    r = r_id[...]
    resp_v = resp[...]
    resp_rows = jnp.where(
        r == 0, resp_v[0:1, :], jnp.where(r == 1, resp_v[1:2, :], resp_v[2:3, :])
    )
    dec_o[...] = resp_rows + et[...] * etW[...] + etb[...] + posv
    out_o[...] = (
        jnp.dot(out_nlp[...], Wv, preferred_element_type=jnp.float32)
        + bv + gexe[...] + gskill[...]
    )


def _tc_fused(in_nlp2, out_nlp2, exe_ids, skill_ids, r_ids, et2,
              g_exe, g_skill, pos_rep, W, b, etW, etb, resp_pad,
              rows_per_block, interpret=False):
    n, nlp = in_nlp2.shape
    d = W.shape[1]
    r_blk = rows_per_block
    grid = (n // r_blk,)
    row_spec = lambda w: pl.BlockSpec((r_blk, w), lambda i: (i, 0))
    full_spec = lambda h, w: pl.BlockSpec((h, w), lambda i: (0, 0))
    return pl.pallas_call(
        _tc_body,
        grid=grid,
        in_specs=[
            row_spec(nlp), row_spec(nlp),
            row_spec(1), row_spec(1), row_spec(1), row_spec(1),
            row_spec(d), row_spec(d),
            full_spec(r_blk, d),
            full_spec(nlp, d),
            full_spec(1, d), full_spec(1, d), full_spec(1, d),
            full_spec(8, d),
        ],
        out_specs=[row_spec(d), row_spec(d), row_spec(d)],
        out_shape=[jax.ShapeDtypeStruct((n, d), jnp.float32)] * 3,
        compiler_params=pltpu.CompilerParams(
            dimension_semantics=("arbitrary",),
        ),
        interpret=interpret,
    )(in_nlp2, out_nlp2, exe_ids, skill_ids, r_ids, et2,
      g_exe, g_skill, pos_rep, W, b, etW, etb, resp_pad)


def kernel(input_nlp_embedding, input_exercise, input_skill, input_r,
           in_elapsed_time, output_nlp_embedding, out_exercise, out_skill,
           exercise_table, skill_table, response_table, pos_table,
           nlp_W, nlp_b, et_W, et_b):
    b_dim, s_dim, nlp = input_nlp_embedding.shape
    d = nlp_W.shape[1]
    n = b_dim * s_dim

    g_exe, g_skill = _sc_gather_pair(
        out_exercise.reshape(n), out_skill.reshape(n),
        exercise_table, skill_table,
    )

    rows_per_block = 400
    pos_rep = jnp.tile(pos_table, (rows_per_block // s_dim, 1))
    resp_pad = jnp.concatenate(
        [response_table,
         jnp.zeros((8 - response_table.shape[0], d), jnp.float32)], axis=0)

    enc, dec, outp = _tc_fused(
        input_nlp_embedding.reshape(n, nlp),
        output_nlp_embedding.reshape(n, nlp),
        input_exercise.reshape(n, 1), input_skill.reshape(n, 1),
        input_r.reshape(n, 1), in_elapsed_time.reshape(n, 1),
        g_exe, g_skill, pos_rep,
        nlp_W, nlp_b.reshape(1, d), et_W.reshape(1, d), et_b.reshape(1, d),
        resp_pad, rows_per_block,
    )
    shape3 = (b_dim, s_dim, d)
    return (enc.reshape(shape3), dec.reshape(shape3), outp.reshape(shape3))


# speedup vs baseline: 1.6309x; 1.6309x over previous
import functools

import jax
import jax.numpy as jnp
from jax import lax
from jax.experimental import pallas as pl
from jax.experimental.pallas import tpu as pltpu
from jax.experimental.pallas import tpu_sc as plsc

_NC = 2
_NS = 16
_NW = _NC * _NS
_CHUNK = 64


def _sc_gather_pair(exe_idx, skill_idx, exe_table, skill_table):
    n = exe_idx.shape[0]
    d = exe_table.shape[1]
    per_w = n // _NW
    n_chunks = per_w // _CHUNK
    mesh = plsc.VectorSubcoreMesh(
        core_axis_name="c", subcore_axis_name="s",
        num_cores=_NC, num_subcores=_NS,
    )

    @functools.partial(
        pl.kernel,
        mesh=mesh,
        out_type=[
            jax.ShapeDtypeStruct((n, d), jnp.float32),
            jax.ShapeDtypeStruct((n, d), jnp.float32),
        ],
        scratch_types=[
            pltpu.VMEM((_CHUNK,), jnp.int32),
            pltpu.VMEM((_CHUNK,), jnp.int32),
            pltpu.VMEM((_CHUNK, d), jnp.float32),
            pltpu.VMEM((_CHUNK, d), jnp.float32),
            pltpu.SemaphoreType.DMA,
            pltpu.SemaphoreType.DMA,
        ],
    )
    def gather_kernel(exe_idx_hbm, skill_idx_hbm, exe_tab_hbm, skill_tab_hbm,
                      out_exe_hbm, out_skill_hbm,
                      idx_e, idx_s, rows_e, rows_s, sem_e, sem_s):
        wid = lax.axis_index("s") * _NC + lax.axis_index("c")
        base = wid * per_w

        def body(c, carry):
            off = base + c * _CHUNK
            pltpu.sync_copy(exe_idx_hbm.at[pl.ds(off, _CHUNK)], idx_e)
            pltpu.sync_copy(skill_idx_hbm.at[pl.ds(off, _CHUNK)], idx_s)
            cp_e = pltpu.async_copy(exe_tab_hbm.at[idx_e], rows_e, sem_e)
            cp_s = pltpu.async_copy(skill_tab_hbm.at[idx_s], rows_s, sem_s)
            cp_e.wait()
            cp_s.wait()
            pltpu.sync_copy(rows_e, out_exe_hbm.at[pl.ds(off, _CHUNK)])
            pltpu.sync_copy(rows_s, out_skill_hbm.at[pl.ds(off, _CHUNK)])
            return carry

        lax.fori_loop(0, n_chunks, body, 0)

    return gather_kernel(exe_idx, skill_idx, exe_table, skill_table)


def _tc_body(bb, in_nlp, out_nlp, exe_id, skill_id, r_id, et, gexe, gskill,
             pos, W, b, etW, etb, resp, enc_o, dec_o, out_o):
    s = pos.shape[0]
    Wv = W[...]
    bv = b[...]
    posv = pos[...]
    respv = resp[...]
    etWv = etW[...]
    etbv = etb[...]
    ones_row = jnp.ones((1, posv.shape[1]), jnp.float32)
    ids2 = (exe_id[...] + skill_id[...]).astype(jnp.float32)
    r2 = r_id[...]
    dn = (((0,), (0,)), ((), ()))
    for j in range(bb):
        ids_bc = lax.dot_general(ids2[j:j + 1, :], ones_row, dn,
                                 preferred_element_type=jnp.float32)
        enc_o[j] = (
            jnp.dot(in_nlp[j], Wv, preferred_element_type=jnp.float32)
            + bv + ids_bc + posv
        )
        rj = r2[j:j + 1, :]
        oh = jnp.concatenate(
            [(rj == t).astype(jnp.float32) for t in range(respv.shape[0])],
            axis=0,
        )
        resp_sel = lax.dot_general(oh, respv, dn,
                                   preferred_element_type=jnp.float32)
        et_bc = jnp.dot(et[j], etWv, preferred_element_type=jnp.float32)
        dec_o[j] = resp_sel + et_bc + etbv + posv
        out_o[j] = (
            jnp.dot(out_nlp[j], Wv, preferred_element_type=jnp.float32)
            + bv + gexe[pl.ds(j * s, s), :] + gskill[pl.ds(j * s, s), :]
        )


def _tc_fused(in_nlp, out_nlp, exe_ids, skill_ids, r_ids, et,
              g_exe, g_skill, pos, W, b, etW, etb, resp,
              bb, interpret=False):
    bsz, s, nlp = in_nlp.shape
    d = W.shape[1]
    grid = (bsz // bb,)
    batch3 = lambda w: pl.BlockSpec((bb, s, w), lambda i: (i, 0, 0))
    batch2 = pl.BlockSpec((bb, s), lambda i: (i, 0))
    rows2 = pl.BlockSpec((bb * s, d), lambda i: (i, 0))
    full2 = lambda h: pl.BlockSpec((h, d), lambda i: (0, 0))
    return pl.pallas_call(
        functools.partial(_tc_body, bb),
        grid=grid,
        in_specs=[
            batch3(nlp), batch3(nlp),
            batch2, batch2, batch2, batch3(1),
            rows2, rows2,
            full2(s),
            pl.BlockSpec((nlp, d), lambda i: (0, 0)),
            full2(1), full2(1), full2(1), full2(resp.shape[0]),
        ],
        out_specs=[batch3(d), batch3(d), batch3(d)],
        out_shape=[jax.ShapeDtypeStruct((bsz, s, d), jnp.float32)] * 3,
        compiler_params=pltpu.CompilerParams(
            dimension_semantics=("arbitrary",),
        ),
        interpret=interpret,
    )(in_nlp, out_nlp, exe_ids, skill_ids, r_ids, et,
      g_exe, g_skill, pos, W, b, etW, etb, resp)


def kernel(input_nlp_embedding, input_exercise, input_skill, input_r,
           in_elapsed_time, output_nlp_embedding, out_exercise, out_skill,
           exercise_table, skill_table, response_table, pos_table,
           nlp_W, nlp_b, et_W, et_b):
    b_dim, s_dim, nlp = input_nlp_embedding.shape
    d = nlp_W.shape[1]
    n = b_dim * s_dim

    g_exe, g_skill = _sc_gather_pair(
        out_exercise.reshape(n), out_skill.reshape(n),
        exercise_table, skill_table,
    )

    enc, dec, outp = _tc_fused(
        input_nlp_embedding, output_nlp_embedding,
        input_exercise, input_skill, input_r, in_elapsed_time,
        g_exe, g_skill, pos_table,
        nlp_W, nlp_b.reshape(1, d), et_W, et_b.reshape(1, d),
        response_table, bb=8,
    )
    return (enc, dec, outp)
